# Initial kernel scaffold; baseline (speedup 1.0000x reference)
import jax
import jax.numpy as jnp
from jax.experimental import pallas as pl


def kernel(feat_f0, feat_f1, feat_c0, feat_c1, b_ids, i_ids, j_ids, Wq, Wkv, Wm, Wm1, Wm2, ke_w0, ke_b0, ke_w1, ke_b1, ke_w2, ke_b2, ke_w3, ke_b3, ke_w4, ke_b4):
    raise NotImplementedError("write your pallas kernel here")


# speedup vs baseline: 2.4525x; 2.4525x over previous
import functools

import jax
import jax.numpy as jnp
from jax import lax
from jax.experimental import pallas as pl
from jax.experimental.pallas import tpu as pltpu
from jax.experimental.pallas import tpu_sc as plsc


def _mm_body(x_ref, w_ref, o_ref):
    o_ref[...] = jnp.dot(x_ref[...], w_ref[...],
                         preferred_element_type=jnp.float32)


def _kv_matmul(x, w_t):
    m, k = x.shape
    blk = 4096
    grid = (m // blk,)
    return pl.pallas_call(
        _mm_body,
        grid=grid,
        in_specs=[
            pl.BlockSpec((blk, k), lambda i: (i, 0)),
            pl.BlockSpec((k, w_t.shape[1]), lambda i: (0, 0)),
        ],
        out_specs=pl.BlockSpec((blk, w_t.shape[1]), lambda i: (i, 0)),
        out_shape=jax.ShapeDtypeStruct((m, w_t.shape[1]), jnp.float32),
    )(x, w_t)


def _pe_body(w0, b0, w1, b1, w2, b2, w3, b3, w4, b4, o_ref):
    k = lax.broadcasted_iota(jnp.int32, (49, 1), 0)
    r = (k % 7 - 3).astype(jnp.float32)
    c = (k // 7 - 3).astype(jnp.float32)
    x = jnp.concatenate([r, c, jnp.zeros((49, 6), jnp.float32)], axis=1)
    x = jnp.dot(x, w0[...], preferred_element_type=jnp.float32) + b0[...]
    x = jnp.maximum(x, 0.0)
    x = jnp.dot(x, w1[...], preferred_element_type=jnp.float32) + b1[...]
    x = jnp.maximum(x, 0.0)
    x = jnp.dot(x, w2[...], preferred_element_type=jnp.float32) + b2[...]
    x = jnp.maximum(x, 0.0)
    x = jnp.dot(x, w3[...], preferred_element_type=jnp.float32) + b3[...]
    x = jnp.maximum(x, 0.0)
    x = jnp.dot(x, w4[...], preferred_element_type=jnp.float32) + b4[...]
    o_ref[...] = x


def _pe_compute(kw, kb):
    w0t = jnp.pad(kw[0].T, ((0, 6), (0, 0)))
    args = [w0t, kb[0][None, :]]
    for i in range(1, 5):
        args += [kw[i].T, kb[i][None, :]]
    return pl.pallas_call(
        _pe_body,
        out_shape=jax.ShapeDtypeStruct((49, 64), jnp.float32),
    )(*args)


_C = 128


def _sc_gather_body(n49, nq, nw, tab_hbm, q0_hbm, idx_hbm, idx3_hbm,
                    og, oq, idx_v, rows_v, sem1):
    nc = 2
    wid = lax.axis_index("s") * nc + lax.axis_index("c")
    per_w = n49 // nw
    base0 = wid * per_w

    def body(i, carry):
        base = base0 + i * _C
        pltpu.sync_copy(idx_hbm.at[pl.ds(base, _C)], idx_v)
        pltpu.async_copy(tab_hbm.at[idx_v], rows_v, sem1).wait()
        pltpu.sync_copy(rows_v, og.at[pl.ds(base, _C)])
        return carry

    lax.fori_loop(0, per_w // _C, body, 0)

    qper = nq // nw
    qb = wid * qper

    def body3(i, carry):
        b = qb + i * _C
        pltpu.sync_copy(idx3_hbm.at[pl.ds(b, _C)], idx_v)
        pltpu.async_copy(q0_hbm.at[idx_v], rows_v, sem1).wait()
        pltpu.sync_copy(rows_v, oq.at[pl.ds(b, _C)])
        return carry

    lax.fori_loop(0, qper // _C, body3, 0)


def _sc_gather(tab, q0p, idx, idx3):
    n49 = idx.shape[0]
    nq = idx3.shape[0]
    info = plsc.get_sparse_core_info()
    nw = info.num_cores * info.num_subcores
    mesh = plsc.VectorSubcoreMesh(core_axis_name="c", subcore_axis_name="s")
    f = functools.partial(
        pl.kernel,
        mesh=mesh,
        out_type=(
            jax.ShapeDtypeStruct((n49, 128), jnp.float32),
            jax.ShapeDtypeStruct((nq, 128), jnp.float32),
        ),
        scratch_types=[
            pltpu.VMEM((_C,), jnp.int32),
            pltpu.VMEM((_C, 128), jnp.float32),
            pltpu.SemaphoreType.DMA,
        ],
    )(functools.partial(_sc_gather_body, n49, nq, nw))
    return f(tab, q0p, idx, idx3)


def _attn_body(fq_ref, g_ref, pe_ref, wq_ref, wm_ref, wm1_ref,
               w2_ref, out0_ref, out1_ref, conf_ref):
    fq = fq_ref[:, :64]
    pe = pe_ref[...]
    g = g_ref[...]
    kv3 = g[:, :, :64] + pe[None]
    out1_ref[...] = g[:, :, 64:] + pe[None]
    q = jnp.dot(fq, wq_ref[...], preferred_element_type=jnp.float32)
    p3 = kv3 * q[:, None, :]
    outs = []
    for h in range(4):
        sl = slice(h * 16, (h + 1) * 16)
        t = jnp.sum(p3[:, :, sl], axis=2) * 0.25
        m = jnp.max(t, axis=1, keepdims=True)
        e = jnp.exp(t - m)
        s = jnp.sum(e, axis=1, keepdims=True)
        a = e / s
        outs.append(jnp.sum(a[:, :, None] * kv3[:, :, sl], axis=1))
    o = jnp.concatenate(outs, axis=1)
    mm = jnp.dot(o, wm_ref[...], preferred_element_type=jnp.float32)
    cat = jnp.concatenate([fq, mm], axis=1)
    h1 = jnp.maximum(
        jnp.dot(cat, wm1_ref[...], preferred_element_type=jnp.float32), 0.0)
    mlp2 = jnp.dot(h1, w2_ref[...], preferred_element_type=jnp.float32)
    out0_ref[...] = mlp2[:, :64] + fq
    conf_ref[...] = jax.nn.sigmoid(mlp2[:, 64:65])


def _attn_call(fqp, g3, pe, wq_t, wm_t, wm1_t, w2p):
    n = fqp.shape[0]
    blk = 128
    grid = (n // blk,)
    return pl.pallas_call(
        _attn_body,
        grid=grid,
        in_specs=[
            pl.BlockSpec((blk, 128), lambda i: (i, 0)),
            pl.BlockSpec((blk, 49, 128), lambda i: (i, 0, 0)),
            pl.BlockSpec((49, 64), lambda i: (0, 0)),
            pl.BlockSpec((64, 64), lambda i: (0, 0)),
            pl.BlockSpec((64, 64), lambda i: (0, 0)),
            pl.BlockSpec((128, 64), lambda i: (0, 0)),
            pl.BlockSpec((64, 128), lambda i: (0, 0)),
        ],
        out_specs=[
            pl.BlockSpec((blk, 64), lambda i: (i, 0)),
            pl.BlockSpec((blk, 49, 64), lambda i: (i, 0, 0)),
            pl.BlockSpec((blk, 1), lambda i: (i, 0)),
        ],
        out_shape=[
            jax.ShapeDtypeStruct((n, 64), jnp.float32),
            jax.ShapeDtypeStruct((n, 49, 64), jnp.float32),
            jax.ShapeDtypeStruct((n, 1), jnp.float32),
        ],
    )(fqp, g3, pe, wq_t, wm_t, wm1_t, w2p)


def kernel(feat_f0, feat_f1, feat_c0, feat_c1, b_ids, i_ids, j_ids, Wq, Wkv,
           Wm, Wm1, Wm2, ke_w0, ke_b0, ke_w1, ke_b1, ke_w2, ke_b2, ke_w3,
           ke_b3, ke_w4, ke_b4):
    b, f, h, w = feat_f0.shape
    hp, wp = h + 6, w + 6
    n = i_ids.shape[0]

    kv = _kv_matmul(feat_f1.reshape(-1, f), Wkv.T)
    kv_pad = jnp.pad(kv.reshape(h, w, f), ((3, 3), (3, 3), (0, 0)))
    kv_pad = kv_pad.reshape(hp * wp, f)

    f1 = jnp.transpose(feat_f1, (0, 2, 3, 1)).reshape(h, w, f)
    f1_pad = jnp.pad(f1, ((3, 3), (3, 3), (0, 0))).reshape(hp * wp, f)

    q0 = jnp.transpose(feat_f0[0, :, 2::4, 2::4], (1, 2, 0))
    q0 = q0.reshape((h // 4) * (w // 4), f)

    kk = jnp.arange(49, dtype=jnp.int32)
    off = (kk % 7 - 3) + (kk // 7 - 3) * wp
    idx = ((j_ids + b_ids * hp * wp)[:, None] + off[None, :]).reshape(-1)
    idx = idx.astype(jnp.int32)
    idx3 = (i_ids + b_ids * (h // 4) * (w // 4)).astype(jnp.int32)

    tab = jnp.concatenate([kv_pad, f1_pad], axis=1)
    q0p = jnp.pad(q0, ((0, 0), (0, 64)))

    g, fqp = _sc_gather(tab, q0p, idx, idx3)
    g3 = g.reshape(n, 49, 128)

    pe = _pe_compute([ke_w0, ke_w1, ke_w2, ke_w3, ke_w4],
                     [ke_b0, ke_b1, ke_b2, ke_b3, ke_b4])

    w2p = jnp.pad(Wm2.T, ((0, 0), (0, 128 - Wm2.shape[0])))
    out0, out1, conf = _attn_call(fqp, g3, pe, Wq.T, Wm.T, Wm1.T, w2p)
    return out0.reshape(n, 1, f), out1, conf.reshape(n)


# speedup vs baseline: 6.1198x; 2.4953x over previous
import functools

import jax
import jax.numpy as jnp
from jax import lax
from jax.experimental import pallas as pl
from jax.experimental.pallas import tpu as pltpu
from jax.experimental.pallas import tpu_sc as plsc


def _mm_body(x_ref, w_ref, o_ref):
    o_ref[...] = jnp.dot(x_ref[...], w_ref[...],
                         preferred_element_type=jnp.float32)


def _kv_matmul(x, w_t):
    m, k = x.shape
    blk = 4096
    grid = (m // blk,)
    return pl.pallas_call(
        _mm_body,
        grid=grid,
        in_specs=[
            pl.BlockSpec((blk, k), lambda i: (i, 0)),
            pl.BlockSpec((k, w_t.shape[1]), lambda i: (0, 0)),
        ],
        out_specs=pl.BlockSpec((blk, w_t.shape[1]), lambda i: (i, 0)),
        out_shape=jax.ShapeDtypeStruct((m, w_t.shape[1]), jnp.float32),
    )(x, w_t)


def _pe_body(w0, b0, w1, b1, w2, b2, w3, b3, w4, b4, o_ref):
    k = lax.broadcasted_iota(jnp.int32, (49, 1), 0)
    r = (k % 7 - 3).astype(jnp.float32)
    c = (k // 7 - 3).astype(jnp.float32)
    x = jnp.concatenate([r, c, jnp.zeros((49, 6), jnp.float32)], axis=1)
    x = jnp.dot(x, w0[...], preferred_element_type=jnp.float32) + b0[...]
    x = jnp.maximum(x, 0.0)
    x = jnp.dot(x, w1[...], preferred_element_type=jnp.float32) + b1[...]
    x = jnp.maximum(x, 0.0)
    x = jnp.dot(x, w2[...], preferred_element_type=jnp.float32) + b2[...]
    x = jnp.maximum(x, 0.0)
    x = jnp.dot(x, w3[...], preferred_element_type=jnp.float32) + b3[...]
    x = jnp.maximum(x, 0.0)
    x = jnp.dot(x, w4[...], preferred_element_type=jnp.float32) + b4[...]
    o_ref[...] = x


def _pe_compute(kw, kb):
    w0t = jnp.pad(kw[0].T, ((0, 6), (0, 0)))
    args = [w0t, kb[0][None, :]]
    for i in range(1, 5):
        args += [kw[i].T, kb[i][None, :]]
    return pl.pallas_call(
        _pe_body,
        out_shape=jax.ShapeDtypeStruct((49, 64), jnp.float32),
    )(*args)


_C = 128


def _sc_gather_body(n49, nq, nw, tab_hbm, q0_hbm, idx_hbm, idx3_hbm,
                    og, oq, idx_v, rows_v, sem1):
    nc = 2
    wid = lax.axis_index("s") * nc + lax.axis_index("c")
    per_w = n49 // nw
    base0 = wid * per_w

    def body(i, carry):
        base = base0 + i * _C
        pltpu.sync_copy(idx_hbm.at[pl.ds(base, _C)], idx_v)
        pltpu.async_copy(tab_hbm.at[idx_v], rows_v, sem1).wait()
        pltpu.sync_copy(rows_v, og.at[pl.ds(base, _C)])
        return carry

    lax.fori_loop(0, per_w // _C, body, 0)

    qper = nq // nw
    qb = wid * qper

    def body3(i, carry):
        b = qb + i * _C
        pltpu.sync_copy(idx3_hbm.at[pl.ds(b, _C)], idx_v)
        pltpu.async_copy(q0_hbm.at[idx_v], rows_v, sem1).wait()
        pltpu.sync_copy(rows_v, oq.at[pl.ds(b, _C)])
        return carry

    lax.fori_loop(0, qper // _C, body3, 0)


def _sc_gather(tab, q0p, idx, idx3):
    n49 = idx.shape[0]
    nq = idx3.shape[0]
    info = plsc.get_sparse_core_info()
    nw = info.num_cores * info.num_subcores
    mesh = plsc.VectorSubcoreMesh(core_axis_name="c", subcore_axis_name="s")
    f = functools.partial(
        pl.kernel,
        mesh=mesh,
        out_type=(
            jax.ShapeDtypeStruct((n49, 128), jnp.float32),
            jax.ShapeDtypeStruct((nq, 128), jnp.float32),
        ),
        scratch_types=[
            pltpu.VMEM((_C,), jnp.int32),
            pltpu.VMEM((_C, 128), jnp.float32),
            pltpu.SemaphoreType.DMA,
        ],
    )(functools.partial(_sc_gather_body, n49, nq, nw))
    return f(tab, q0p, idx, idx3)


def _attn_body(fq_ref, g_ref, pe_ref, wq_ref, wm_ref, wm1_ref,
               w2_ref, sh_ref, eh_ref, out0_ref, out1_ref, conf_ref):
    blk = fq_ref.shape[0]
    fq = fq_ref[:, :64]
    pe = pe_ref[...]
    g = g_ref[...]
    kv3 = g[:, :, :64] + pe[:, None, :]
    out1_ref[...] = g[:, :, 64:] + pe[:, None, :]
    q = jnp.dot(fq, wq_ref[...], preferred_element_type=jnp.float32)
    p3 = kv3 * q[None, :, :]
    qk = jnp.dot(p3.reshape(49 * blk, 64), sh_ref[...],
                 preferred_element_type=jnp.float32)
    t3 = qk.reshape(49, blk, 128) * 0.25
    m = jnp.max(t3, axis=0)
    e3 = jnp.exp(t3 - m[None])
    ssum = jnp.sum(e3, axis=0)
    a3 = e3 / ssum[None]
    ae = jnp.dot(a3.reshape(49 * blk, 128), eh_ref[...],
                 preferred_element_type=jnp.float32)
    o = jnp.sum(ae.reshape(49, blk, 64) * kv3, axis=0)
    mm = jnp.dot(o, wm_ref[...], preferred_element_type=jnp.float32)
    cat = jnp.concatenate([fq, mm], axis=1)
    h1 = jnp.maximum(
        jnp.dot(cat, wm1_ref[...], preferred_element_type=jnp.float32), 0.0)
    mlp2 = jnp.dot(h1, w2_ref[...], preferred_element_type=jnp.float32)
    out0_ref[...] = mlp2[:, :64] + fq
    conf_ref[...] = jax.nn.sigmoid(mlp2[:, 64:65])


def _attn_call(fqp, g3, pe, wq_t, wm_t, wm1_t, w2p):
    n = fqp.shape[0]
    blk = 256
    grid = (n // blk,)
    ii = jnp.arange(64)[:, None] // 16
    sh = (ii == jnp.arange(128)[None, :]).astype(jnp.float32)
    eh = sh.T
    return pl.pallas_call(
        _attn_body,
        grid=grid,
        in_specs=[
            pl.BlockSpec((blk, 128), lambda i: (i, 0)),
            pl.BlockSpec((49, blk, 128), lambda i: (0, i, 0)),
            pl.BlockSpec((49, 64), lambda i: (0, 0)),
            pl.BlockSpec((64, 64), lambda i: (0, 0)),
            pl.BlockSpec((64, 64), lambda i: (0, 0)),
            pl.BlockSpec((128, 64), lambda i: (0, 0)),
            pl.BlockSpec((64, 128), lambda i: (0, 0)),
            pl.BlockSpec((64, 128), lambda i: (0, 0)),
            pl.BlockSpec((128, 64), lambda i: (0, 0)),
        ],
        out_specs=[
            pl.BlockSpec((blk, 64), lambda i: (i, 0)),
            pl.BlockSpec((49, blk, 64), lambda i: (0, i, 0)),
            pl.BlockSpec((blk, 1), lambda i: (i, 0)),
        ],
        out_shape=[
            jax.ShapeDtypeStruct((n, 64), jnp.float32),
            jax.ShapeDtypeStruct((49, n, 64), jnp.float32),
            jax.ShapeDtypeStruct((n, 1), jnp.float32),
        ],
    )(fqp, g3, pe, wq_t, wm_t, wm1_t, w2p, sh, eh)


def kernel(feat_f0, feat_f1, feat_c0, feat_c1, b_ids, i_ids, j_ids, Wq, Wkv,
           Wm, Wm1, Wm2, ke_w0, ke_b0, ke_w1, ke_b1, ke_w2, ke_b2, ke_w3,
           ke_b3, ke_w4, ke_b4):
    b, f, h, w = feat_f0.shape
    hp, wp = h + 6, w + 6
    n = i_ids.shape[0]

    kv = _kv_matmul(feat_f1.reshape(-1, f), Wkv.T)
    kv_pad = jnp.pad(kv.reshape(h, w, f), ((3, 3), (3, 3), (0, 0)))
    kv_pad = kv_pad.reshape(hp * wp, f)

    f1 = jnp.transpose(feat_f1, (0, 2, 3, 1)).reshape(h, w, f)
    f1_pad = jnp.pad(f1, ((3, 3), (3, 3), (0, 0))).reshape(hp * wp, f)

    q0 = jnp.transpose(feat_f0[0, :, 2::4, 2::4], (1, 2, 0))
    q0 = q0.reshape((h // 4) * (w // 4), f)

    kk = jnp.arange(49, dtype=jnp.int32)
    off = (kk % 7 - 3) + (kk // 7 - 3) * wp
    idx = (off[:, None] + (j_ids + b_ids * hp * wp)[None, :]).reshape(-1)
    idx = idx.astype(jnp.int32)
    idx3 = (i_ids + b_ids * (h // 4) * (w // 4)).astype(jnp.int32)

    tab = jnp.concatenate([kv_pad, f1_pad], axis=1)
    q0p = jnp.pad(q0, ((0, 0), (0, 64)))

    g, fqp = _sc_gather(tab, q0p, idx, idx3)
    g3 = g.reshape(49, n, 128)

    pe = _pe_compute([ke_w0, ke_w1, ke_w2, ke_w3, ke_w4],
                     [ke_b0, ke_b1, ke_b2, ke_b3, ke_b4])

    w2p = jnp.pad(Wm2.T, ((0, 0), (0, 128 - Wm2.shape[0])))
    out0, out1s, conf = _attn_call(fqp, g3, pe, Wq.T, Wm.T, Wm1.T, w2p)
    out1 = jnp.transpose(out1s, (1, 0, 2))
    return out0.reshape(n, 1, f), out1, conf.reshape(n)


# speedup vs baseline: 7.0618x; 1.1539x over previous
import functools

import jax
import jax.numpy as jnp
from jax import lax
from jax.experimental import pallas as pl
from jax.experimental.pallas import tpu as pltpu
from jax.experimental.pallas import tpu_sc as plsc


def _mm_body(x_ref, w_ref, o_ref):
    o_ref[...] = jnp.dot(x_ref[...], w_ref[...],
                         preferred_element_type=jnp.float32)


def _kv_matmul(x, w_t):
    m, k = x.shape
    blk = 4096
    grid = (m // blk,)
    return pl.pallas_call(
        _mm_body,
        grid=grid,
        in_specs=[
            pl.BlockSpec((blk, k), lambda i: (i, 0)),
            pl.BlockSpec((k, w_t.shape[1]), lambda i: (0, 0)),
        ],
        out_specs=pl.BlockSpec((blk, w_t.shape[1]), lambda i: (i, 0)),
        out_shape=jax.ShapeDtypeStruct((m, w_t.shape[1]), jnp.float32),
    )(x, w_t)


def _pe_body(w0, b0, w1, b1, w2, b2, w3, b3, w4, b4, o_ref):
    k = lax.broadcasted_iota(jnp.int32, (49, 1), 0)
    r = (k % 7 - 3).astype(jnp.float32)
    c = (k // 7 - 3).astype(jnp.float32)
    x = jnp.concatenate([r, c, jnp.zeros((49, 6), jnp.float32)], axis=1)
    x = jnp.dot(x, w0[...], preferred_element_type=jnp.float32) + b0[...]
    x = jnp.maximum(x, 0.0)
    x = jnp.dot(x, w1[...], preferred_element_type=jnp.float32) + b1[...]
    x = jnp.maximum(x, 0.0)
    x = jnp.dot(x, w2[...], preferred_element_type=jnp.float32) + b2[...]
    x = jnp.maximum(x, 0.0)
    x = jnp.dot(x, w3[...], preferred_element_type=jnp.float32) + b3[...]
    x = jnp.maximum(x, 0.0)
    x = jnp.dot(x, w4[...], preferred_element_type=jnp.float32) + b4[...]
    o_ref[...] = x


def _pe_compute(kw, kb):
    w0t = jnp.pad(kw[0].T, ((0, 6), (0, 0)))
    args = [w0t, kb[0][None, :]]
    for i in range(1, 5):
        args += [kw[i].T, kb[i][None, :]]
    return pl.pallas_call(
        _pe_body,
        out_shape=jax.ShapeDtypeStruct((49, 64), jnp.float32),
    )(*args)


_C = 128


_CW = 112
_NB = 4


def _sc_gather_body(n49, nq, nw, tab_hbm, q0_hbm, idx_hbm, idx3_hbm,
                    og, oq, idx_v, r0, r1, r2, r3, gsem, wsem):
    nc = 2
    wid = lax.axis_index("s") * nc + lax.axis_index("c")
    per_w = n49 // nw
    base0 = wid * per_w
    nblk = per_w // (_NB * _CW)
    bufs = [r0, r1, r2, r3]

    pltpu.sync_copy(idx_hbm.at[pl.ds(base0, per_w)], idx_v)

    def blk_body(b, carry):
        @pl.when(b > 0)
        def _drain():
            for j in range(_NB):
                pltpu.make_async_copy(
                    og.at[pl.ds(base0, _CW)], bufs[j].at[pl.ds(0, _CW)],
                    wsem).wait()
        hs = []
        for j in range(_NB):
            k = b * _NB + j
            hs.append(pltpu.async_copy(
                tab_hbm.at[idx_v.at[pl.ds(k * _CW, _CW)]],
                bufs[j].at[pl.ds(0, _CW)], gsem))
        for h in hs:
            h.wait()
        for j in range(_NB):
            k = b * _NB + j
            pltpu.async_copy(bufs[j].at[pl.ds(0, _CW)],
                             og.at[pl.ds(base0 + k * _CW, _CW)], wsem)
        return carry

    lax.fori_loop(0, nblk, blk_body, 0)
    for j in range(_NB):
        pltpu.make_async_copy(og.at[pl.ds(base0, _CW)],
                              bufs[j].at[pl.ds(0, _CW)], wsem).wait()

    qper = nq // nw
    qb = wid * qper

    def body3(i, carry):
        b = qb + i * _C
        pltpu.sync_copy(idx3_hbm.at[pl.ds(b, _C)], idx_v.at[pl.ds(0, _C)])
        pltpu.async_copy(q0_hbm.at[idx_v.at[pl.ds(0, _C)]], r0, gsem).wait()
        pltpu.sync_copy(r0, oq.at[pl.ds(b, _C)])
        return carry

    lax.fori_loop(0, qper // _C, body3, 0)


def _sc_gather(tab, q0p, idx, idx3):
    n49 = idx.shape[0]
    nq = idx3.shape[0]
    info = plsc.get_sparse_core_info()
    nw = info.num_cores * info.num_subcores
    per_w = n49 // nw
    mesh = plsc.VectorSubcoreMesh(core_axis_name="c", subcore_axis_name="s")
    f = functools.partial(
        pl.kernel,
        mesh=mesh,
        out_type=(
            jax.ShapeDtypeStruct((n49, 128), jnp.float32),
            jax.ShapeDtypeStruct((nq, 128), jnp.float32),
        ),
        scratch_types=[
            pltpu.VMEM((per_w,), jnp.int32),
            pltpu.VMEM((_C, 128), jnp.float32),
            pltpu.VMEM((_C, 128), jnp.float32),
            pltpu.VMEM((_C, 128), jnp.float32),
            pltpu.VMEM((_C, 128), jnp.float32),
            pltpu.SemaphoreType.DMA,
            pltpu.SemaphoreType.DMA,
        ],
    )(functools.partial(_sc_gather_body, n49, nq, nw))
    return f(tab, q0p, idx, idx3)


def _attn_body(fq_ref, g_ref, pe_ref, wq_ref, wm_ref, wm1_ref,
               w2_ref, sh_ref, eh_ref, out0_ref, out1_ref, conf_ref):
    blk = fq_ref.shape[0]
    fq = fq_ref[:, :64]
    pe = pe_ref[...]
    g = g_ref[...]
    kv3 = g[:, :, :64] + pe[:, None, :]
    out1_ref[...] = g[:, :, 64:] + pe[:, None, :]
    q = jnp.dot(fq, wq_ref[...], preferred_element_type=jnp.float32)
    p3 = kv3 * q[None, :, :]
    qk = jnp.dot(p3.reshape(49 * blk, 64), sh_ref[...],
                 preferred_element_type=jnp.float32)
    t3 = qk.reshape(49, blk, 128) * 0.25
    m = jnp.max(t3, axis=0)
    e3 = jnp.exp(t3 - m[None])
    ssum = jnp.sum(e3, axis=0)
    a3 = e3 / ssum[None]
    ae = jnp.dot(a3.reshape(49 * blk, 128), eh_ref[...],
                 preferred_element_type=jnp.float32)
    o = jnp.sum(ae.reshape(49, blk, 64) * kv3, axis=0)
    mm = jnp.dot(o, wm_ref[...], preferred_element_type=jnp.float32)
    cat = jnp.concatenate([fq, mm], axis=1)
    h1 = jnp.maximum(
        jnp.dot(cat, wm1_ref[...], preferred_element_type=jnp.float32), 0.0)
    mlp2 = jnp.dot(h1, w2_ref[...], preferred_element_type=jnp.float32)
    out0_ref[...] = mlp2[:, :64] + fq
    conf_ref[...] = jax.nn.sigmoid(mlp2[:, 64:65])


def _attn_call(fqp, g3, pe, wq_t, wm_t, wm1_t, w2p):
    n = fqp.shape[0]
    blk = 256
    grid = (n // blk,)
    ii = jnp.arange(64)[:, None] // 16
    sh = (ii == jnp.arange(128)[None, :]).astype(jnp.float32)
    eh = sh.T
    return pl.pallas_call(
        _attn_body,
        grid=grid,
        in_specs=[
            pl.BlockSpec((blk, 128), lambda i: (i, 0)),
            pl.BlockSpec((49, blk, 128), lambda i: (0, i, 0)),
            pl.BlockSpec((49, 64), lambda i: (0, 0)),
            pl.BlockSpec((64, 64), lambda i: (0, 0)),
            pl.BlockSpec((64, 64), lambda i: (0, 0)),
            pl.BlockSpec((128, 64), lambda i: (0, 0)),
            pl.BlockSpec((64, 128), lambda i: (0, 0)),
            pl.BlockSpec((64, 128), lambda i: (0, 0)),
            pl.BlockSpec((128, 64), lambda i: (0, 0)),
        ],
        out_specs=[
            pl.BlockSpec((blk, 64), lambda i: (i, 0)),
            pl.BlockSpec((49, blk, 64), lambda i: (0, i, 0)),
            pl.BlockSpec((blk, 1), lambda i: (i, 0)),
        ],
        out_shape=[
            jax.ShapeDtypeStruct((n, 64), jnp.float32),
            jax.ShapeDtypeStruct((49, n, 64), jnp.float32),
            jax.ShapeDtypeStruct((n, 1), jnp.float32),
        ],
    )(fqp, g3, pe, wq_t, wm_t, wm1_t, w2p, sh, eh)


def kernel(feat_f0, feat_f1, feat_c0, feat_c1, b_ids, i_ids, j_ids, Wq, Wkv,
           Wm, Wm1, Wm2, ke_w0, ke_b0, ke_w1, ke_b1, ke_w2, ke_b2, ke_w3,
           ke_b3, ke_w4, ke_b4):
    b, f, h, w = feat_f0.shape
    hp, wp = h + 6, w + 6
    n = i_ids.shape[0]

    kv = _kv_matmul(feat_f1.reshape(-1, f), Wkv.T)
    kv_pad = jnp.pad(kv.reshape(h, w, f), ((3, 3), (3, 3), (0, 0)))
    kv_pad = kv_pad.reshape(hp * wp, f)

    f1 = jnp.transpose(feat_f1, (0, 2, 3, 1)).reshape(h, w, f)
    f1_pad = jnp.pad(f1, ((3, 3), (3, 3), (0, 0))).reshape(hp * wp, f)

    q0 = jnp.transpose(feat_f0[0, :, 2::4, 2::4], (1, 2, 0))
    q0 = q0.reshape((h // 4) * (w // 4), f)

    kk = jnp.arange(49, dtype=jnp.int32)
    off = (kk % 7 - 3) + (kk // 7 - 3) * wp
    idx = (off[:, None] + (j_ids + b_ids * hp * wp)[None, :]).reshape(-1)
    idx = idx.astype(jnp.int32)
    idx3 = (i_ids + b_ids * (h // 4) * (w // 4)).astype(jnp.int32)

    tab = jnp.concatenate([kv_pad, f1_pad], axis=1)
    q0p = jnp.pad(q0, ((0, 0), (0, 64)))

    g, fqp = _sc_gather(tab, q0p, idx, idx3)
    g3 = g.reshape(49, n, 128)

    pe = _pe_compute([ke_w0, ke_w1, ke_w2, ke_w3, ke_w4],
                     [ke_b0, ke_b1, ke_b2, ke_b3, ke_b4])

    w2p = jnp.pad(Wm2.T, ((0, 0), (0, 128 - Wm2.shape[0])))
    out0, out1s, conf = _attn_call(fqp, g3, pe, Wq.T, Wm.T, Wm1.T, w2p)
    out1 = jnp.transpose(out1s, (1, 0, 2))
    return out0.reshape(n, 1, f), out1, conf.reshape(n)
